# baseline (speedup 1.0000x reference)
import jax
import jax.numpy as jnp
from jax import lax
from jax.experimental import pallas as pl
from jax.experimental.pallas import tpu as pltpu
from jax.experimental.pallas import tpu_sc as plsc

E = 800000
IN_DIM = 7
EMBED_DIM = 300
DPAD = 304
NJ = DPAD // 16
NW = 32
ROWS_PER_W = E // NW
IB = 1000
C = 200
PC = 40
RG = 8
SB = 4
NIB = ROWS_PER_W // IB
NCI = IB // C


def _full(v):
    return jnp.full((16,), v, jnp.int32)


def _lane(v, i):
    return jnp.take_along_axis(v, _full(i), axis=0,
                               mode="promise_in_bounds")


def _sc_kernel_body(t_hbm, w_hbm, b_hbm, out_hbm,
                    in_buf, out_buf, shared, w_raw, wt_buf, b_buf,
                    sem_a, sem_b):
    wid = lax.axis_index("s") * 2 + lax.axis_index("c")
    row0 = wid * ROWS_PER_W
    iota = lax.iota(jnp.int32, 16)

    pltpu.sync_copy(w_hbm, w_raw.at[pl.ds(0, EMBED_DIM * IN_DIM)])
    pltpu.sync_copy(b_hbm, b_buf.at[pl.ds(0, EMBED_DIM)])
    for k in range(IN_DIM):
        for j in range(NJ):
            d0 = 16 * j if j < NJ - 1 else EMBED_DIM - 16
            col = plsc.load_gather(w_raw, [(d0 + iota) * IN_DIM + k])
            wt_buf[pl.ds(k * DPAD + 16 * j, 16)] = col
    btail = plsc.load_gather(b_buf, [EMBED_DIM - 16 + iota])
    b_buf[pl.ds(DPAD - 16, 16)] = btail

    def compute_rows(lo, hi, roff):
        def group_body(g, carry2):
            r0 = g * RG
            vk = [plsc.load_gather(
                      in_buf, [(roff + r0 + iota) * IN_DIM + k])
                  for k in range(IN_DIM)]
            for s in range(RG // SB):
                splats = [[_lane(vk[k], s * SB + i) for k in range(IN_DIM)]
                          for i in range(SB)]
                for j in range(NJ):
                    bj = b_buf[pl.ds(16 * j, 16)]
                    ws = [wt_buf[pl.ds(k * DPAD + 16 * j, 16)]
                          for k in range(IN_DIM)]
                    for i in range(SB):
                        acc = bj
                        for k in range(IN_DIM):
                            acc = acc + splats[i][k] * ws[k]
                        row = r0 + s * SB + i
                        if j < NJ - 1:
                            out_buf[row, pl.ds(16 * j, 16)] = acc
                        else:
                            plsc.store_scatter(
                                out_buf,
                                [_full(row), EMBED_DIM - 16 + iota], acc)
            return carry2

        lax.fori_loop(lo // RG, hi // RG, group_body, 0)

    sid = lax.axis_index("s")

    def ib_body(ib, carry):
        ib_base = row0 + ib * IB
        pltpu.sync_copy(t_hbm.at[pl.ds(ib_base * IN_DIM, IB * IN_DIM)],
                        in_buf.at[pl.ds(0, IB * IN_DIM)])

        def ci_body(ci, carry2):
            base = ib_base + ci * C
            gc = ib * NCI + ci

            @pl.when(gc > 0)
            def _():
                pltpu.make_async_copy(
                    shared.at[sid], out_hbm.at[pl.ds(row0, C), :],
                    sem_a).wait()

            for p in range(C // PC):
                compute_rows(0, PC, ci * C + p * PC)
                pltpu.sync_copy(out_buf,
                                shared.at[sid, pl.ds(p * PC, PC), :])

            pltpu.async_copy(shared.at[sid],
                             out_hbm.at[pl.ds(base, C), :], sem_a)
            return carry2

        lax.fori_loop(0, NCI, ci_body, 0)
        return carry

    lax.fori_loop(0, NIB, ib_body, 0)
    pltpu.make_async_copy(
        shared.at[sid], out_hbm.at[pl.ds(row0, C), :], sem_a).wait()


def kernel(tensor, W, b):
    mesh = plsc.VectorSubcoreMesh(core_axis_name="c", subcore_axis_name="s")
    return pl.kernel(
        _sc_kernel_body,
        mesh=mesh,
        compiler_params=pltpu.CompilerParams(needs_layout_passes=False),
        out_type=jax.ShapeDtypeStruct((E, EMBED_DIM), jnp.float32),
        scratch_types=[
            pltpu.VMEM((IB * IN_DIM + 56,), jnp.float32),
            pltpu.VMEM((PC, EMBED_DIM), jnp.float32),
            pltpu.VMEM_SHARED((16, C, EMBED_DIM), jnp.float32),
            pltpu.VMEM((EMBED_DIM * IN_DIM + 28,), jnp.float32),
            pltpu.VMEM((IN_DIM * DPAD,), jnp.float32),
            pltpu.VMEM((DPAD,), jnp.float32),
            pltpu.SemaphoreType.DMA,
            pltpu.SemaphoreType.DMA,
        ],
    )(tensor.reshape(E * IN_DIM), W.reshape(EMBED_DIM * IN_DIM), b)


# speedup vs baseline: 1.0278x; 1.0278x over previous
import jax
import jax.numpy as jnp
from jax import lax
from jax.experimental import pallas as pl
from jax.experimental.pallas import tpu as pltpu
from jax.experimental.pallas import tpu_sc as plsc

E = 800000
IN_DIM = 7
EMBED_DIM = 300
DPAD = 304
NJ = DPAD // 16
NW = 32
ROWS_PER_W = E // NW
IB = 1000
C = 200
PC = 40
RG = 8
SB = 4
NIB = ROWS_PER_W // IB
NCI = IB // C


def _full(v):
    return jnp.full((16,), v, jnp.int32)


def _lane(v, i):
    return jnp.take_along_axis(v, _full(i), axis=0,
                               mode="promise_in_bounds")


def _sc_kernel_body(t_hbm, w_hbm, b_hbm, out_hbm,
                    in_buf, out_buf, w_raw, wt_buf, b_buf, *sems):
    wid = lax.axis_index("s") * 2 + lax.axis_index("c")
    row0 = wid * ROWS_PER_W
    iota = lax.iota(jnp.int32, 16)

    pltpu.sync_copy(w_hbm, w_raw.at[pl.ds(0, EMBED_DIM * IN_DIM)])
    pltpu.sync_copy(b_hbm, b_buf.at[pl.ds(0, EMBED_DIM)])
    for k in range(IN_DIM):
        for j in range(NJ):
            d0 = 16 * j if j < NJ - 1 else EMBED_DIM - 16
            col = plsc.load_gather(w_raw, [(d0 + iota) * IN_DIM + k])
            wt_buf[pl.ds(k * DPAD + 16 * j, 16)] = col
    btail = plsc.load_gather(b_buf, [EMBED_DIM - 16 + iota])
    b_buf[pl.ds(DPAD - 16, 16)] = btail

    def compute_rows(lo, hi, roff):
        def group_body(g, carry2):
            r0 = g * RG
            vk = [plsc.load_gather(
                      in_buf, [(roff + r0 + iota) * IN_DIM + k])
                  for k in range(IN_DIM)]
            for s in range(RG // SB):
                splats = [[_lane(vk[k], s * SB + i) for k in range(IN_DIM)]
                          for i in range(SB)]
                for j in range(NJ):
                    bj = b_buf[pl.ds(16 * j, 16)]
                    ws = [wt_buf[pl.ds(k * DPAD + 16 * j, 16)]
                          for k in range(IN_DIM)]
                    for i in range(SB):
                        acc = bj
                        for k in range(IN_DIM):
                            acc = acc + splats[i][k] * ws[k]
                        row = r0 + s * SB + i
                        if j < NJ - 1:
                            out_buf[row, pl.ds(16 * j, 16)] = acc
                        else:
                            plsc.store_scatter(
                                out_buf,
                                [_full(row), EMBED_DIM - 16 + iota], acc)
            return carry2

        lax.fori_loop(lo // RG, hi // RG, group_body, 0)

    def piece(base, p, sem):
        return (out_buf.at[pl.ds(p * PC, PC), :],
                out_hbm.at[pl.ds(base + p * PC, PC), :], sem)

    def ib_body(ib, carry):
        ib_base = row0 + ib * IB
        pltpu.sync_copy(t_hbm.at[pl.ds(ib_base * IN_DIM, IB * IN_DIM)],
                        in_buf.at[pl.ds(0, IB * IN_DIM)])

        def ci_body(ci, carry2):
            base = ib_base + ci * C
            gc = ib * NCI + ci
            roff = ci * C

            for p in range(C // PC):
                @pl.when(gc > 0)
                def _():
                    src, dst, sem = piece(row0, p, sems[p])
                    pltpu.make_async_copy(src, dst, sem).wait()

                compute_rows(p * PC, (p + 1) * PC, roff)
                src, dst, sem = piece(base, p, sems[p])
                pltpu.async_copy(src, dst, sem)
            return carry2

        lax.fori_loop(0, NCI, ci_body, 0)
        return carry

    lax.fori_loop(0, NIB, ib_body, 0)
    for p in range(C // PC):
        src, dst, sem = piece(row0, p, sems[p])
        pltpu.make_async_copy(src, dst, sem).wait()


def kernel(tensor, W, b):
    mesh = plsc.VectorSubcoreMesh(core_axis_name="c", subcore_axis_name="s")
    return pl.kernel(
        _sc_kernel_body,
        mesh=mesh,
        compiler_params=pltpu.CompilerParams(needs_layout_passes=False),
        out_type=jax.ShapeDtypeStruct((E, EMBED_DIM), jnp.float32),
        scratch_types=[
            pltpu.VMEM((IB * IN_DIM + 56,), jnp.float32),
            pltpu.VMEM((C, EMBED_DIM), jnp.float32),
            pltpu.VMEM((EMBED_DIM * IN_DIM + 28,), jnp.float32),
            pltpu.VMEM((IN_DIM * DPAD,), jnp.float32),
            pltpu.VMEM((DPAD,), jnp.float32),
        ] + [pltpu.SemaphoreType.DMA] * (C // PC),
    )(tensor.reshape(E * IN_DIM), W.reshape(EMBED_DIM * IN_DIM), b)


# speedup vs baseline: 1.3172x; 1.2816x over previous
import jax
import jax.numpy as jnp
from jax import lax
from jax.experimental import pallas as pl
from jax.experimental.pallas import tpu as pltpu
from jax.experimental.pallas import tpu_sc as plsc

E = 800000
IN_DIM = 7
EMBED_DIM = 300
DPAD = 304
NJ = DPAD // 16
NW = 32
NS = 5
ES = E // NS
ROWS_PER_W = ES // NW
IB = 1000
C = 200
HA = 96
RG = 8
SB = 4
NIB = ROWS_PER_W // IB
NCI = IB // C


def _full(v):
    return jnp.full((16,), v, jnp.int32)


def _lane(v, i):
    return jnp.take_along_axis(v, _full(i), axis=0,
                               mode="promise_in_bounds")


def _sc_kernel_body(t_hbm, w_hbm, b_hbm, out_hbm,
                    in_buf, out_buf, w_raw, wt_buf, b_buf, sem_a, sem_b):
    wid = lax.axis_index("s") * 2 + lax.axis_index("c")
    row0 = wid * ROWS_PER_W
    iota = lax.iota(jnp.int32, 16)

    pltpu.sync_copy(w_hbm, w_raw.at[pl.ds(0, EMBED_DIM * IN_DIM)])
    pltpu.sync_copy(b_hbm, b_buf.at[pl.ds(0, EMBED_DIM)])
    for k in range(IN_DIM):
        for j in range(NJ):
            d0 = 16 * j if j < NJ - 1 else EMBED_DIM - 16
            col = plsc.load_gather(w_raw, [(d0 + iota) * IN_DIM + k])
            wt_buf[pl.ds(k * DPAD + 16 * j, 16)] = col
    btail = plsc.load_gather(b_buf, [EMBED_DIM - 16 + iota])
    b_buf[pl.ds(DPAD - 16, 16)] = btail

    def compute_rows(lo, hi, roff):
        def group_body(g, carry2):
            r0 = g * RG
            vk = [plsc.load_gather(
                      in_buf, [(roff + r0 + iota) * IN_DIM + k])
                  for k in range(IN_DIM)]
            for s in range(RG // SB):
                splats = [[_lane(vk[k], s * SB + i) for k in range(IN_DIM)]
                          for i in range(SB)]
                for j in range(NJ):
                    bj = b_buf[pl.ds(16 * j, 16)]
                    ws = [wt_buf[pl.ds(k * DPAD + 16 * j, 16)]
                          for k in range(IN_DIM)]
                    for i in range(SB):
                        acc = bj
                        for k in range(IN_DIM):
                            acc = acc + splats[i][k] * ws[k]
                        row = r0 + s * SB + i
                        if j < NJ - 1:
                            out_buf[row, pl.ds(16 * j, 16)] = acc
                        else:
                            plsc.store_scatter(
                                out_buf,
                                [_full(row), EMBED_DIM - 16 + iota], acc)
            return carry2

        lax.fori_loop(lo // RG, hi // RG, group_body, 0)

    def half_a(base):
        return (out_buf.at[pl.ds(0, HA), :],
                out_hbm.at[pl.ds(base, HA), :], sem_a)

    def half_b(base):
        return (out_buf.at[pl.ds(HA, C - HA), :],
                out_hbm.at[pl.ds(base + HA, C - HA), :], sem_b)

    def ib_body(ib, carry):
        ib_base = row0 + ib * IB
        pltpu.sync_copy(t_hbm.at[pl.ds(ib_base * IN_DIM, IB * IN_DIM)],
                        in_buf.at[pl.ds(0, IB * IN_DIM)])

        def ci_body(ci, carry2):
            base = ib_base + ci * C
            gc = ib * NCI + ci
            roff = ci * C

            @pl.when(gc > 0)
            def _():
                src, dst, sem = half_a(row0)
                pltpu.make_async_copy(src, dst, sem).wait()

            compute_rows(0, HA, roff)
            src, dst, sem = half_a(base)
            pltpu.async_copy(src, dst, sem)

            @pl.when(gc > 0)
            def _():
                src2, dst2, sem2 = half_b(row0)
                pltpu.make_async_copy(src2, dst2, sem2).wait()

            compute_rows(HA, C, roff)
            src2, dst2, sem2 = half_b(base)
            pltpu.async_copy(src2, dst2, sem2)
            return carry2

        lax.fori_loop(0, NCI, ci_body, 0)
        return carry

    lax.fori_loop(0, NIB, ib_body, 0)
    src, dst, sem = half_a(row0)
    pltpu.make_async_copy(src, dst, sem).wait()
    src2, dst2, sem2 = half_b(row0)
    pltpu.make_async_copy(src2, dst2, sem2).wait()


def kernel(tensor, W, b):
    mesh = plsc.VectorSubcoreMesh(core_axis_name="c", subcore_axis_name="s")
    stripe = pl.kernel(
        _sc_kernel_body,
        mesh=mesh,
        compiler_params=pltpu.CompilerParams(needs_layout_passes=False),
        out_type=jax.ShapeDtypeStruct((ES, EMBED_DIM), jnp.float32),
        scratch_types=[
            pltpu.VMEM((IB * IN_DIM + 56,), jnp.float32),
            pltpu.VMEM((C, EMBED_DIM), jnp.float32),
            pltpu.VMEM((EMBED_DIM * IN_DIM + 28,), jnp.float32),
            pltpu.VMEM((IN_DIM * DPAD,), jnp.float32),
            pltpu.VMEM((DPAD,), jnp.float32),
            pltpu.SemaphoreType.DMA,
            pltpu.SemaphoreType.DMA,
        ],
    )
    t_flat = tensor.reshape(E * IN_DIM)
    w_flat = W.reshape(EMBED_DIM * IN_DIM)
    outs = [stripe(lax.dynamic_slice(t_flat, (s * ES * IN_DIM,),
                                     (ES * IN_DIM,)), w_flat, b)
            for s in range(NS)]
    return jnp.concatenate(outs, axis=0)
